# baseline (speedup 1.0000x reference)
import jax
import jax.numpy as jnp
from jax.experimental import pallas as pl

TN = 1024
RG = 128


def _rowsum_sq(v):
    d = v.shape[1]
    vk = v[:, 0:128]
    q = vk * vk
    for k in range(1, d // 128):
        vk = v[:, 128 * k:128 * (k + 1)]
        q = q + vk * vk
    a = q[:, 0:8]
    for t in range(1, 16):
        a = a + q[:, 8 * t:8 * (t + 1)]
    b = a[:, 0:4] + a[:, 4:8]
    c = b[:, 0:2] + b[:, 2:4]
    return c[:, 0:1] + c[:, 1:2]


def _rowsum_sq_grouped(v):
    r = v.shape[0]
    if r <= RG:
        return _rowsum_sq(v)
    parts = [_rowsum_sq(v[i:i + RG, :]) for i in range(0, r, RG)]
    return jnp.concatenate(parts, axis=0)


def _c2_body(c_ref, o_ref):
    o_ref[...] = _rowsum_sq(c_ref[...])


def _router_body(x_ref, cb_ref, c2_ref, w_ref, a_ref):
    x = x_ref[...]
    cb = cb_ref[...]
    dot = jax.lax.dot_general(
        x, cb, (((1,), (1,)), ((), ())),
        preferred_element_type=jnp.float32,
    )
    x2 = _rowsum_sq_grouped(x)
    c2 = c2_ref[...]
    sq = jnp.maximum(x2 + c2 - 2.0 * dot, 0.0)
    dists = jnp.sqrt(sq)

    neg = -dists
    m = jnp.max(neg, axis=-1, keepdims=True)
    e = jnp.exp(neg - m)
    w_ref[...] = e / jnp.sum(e, axis=-1, keepdims=True)

    k = dists.shape[-1]
    idx = jax.lax.broadcasted_iota(jnp.int32, dists.shape, 1)
    minv = jnp.min(dists, axis=-1, keepdims=True)
    cand = jnp.where(dists == minv, idx, k)
    a_ref[...] = jnp.min(cand, axis=-1, keepdims=True)


def kernel(x, centroids):
    b, t, d = x.shape
    k = centroids.shape[0]
    n = b * t
    x_flat = x.reshape(n, d)
    c_bf16 = centroids.astype(jnp.bfloat16)

    c2_col = pl.pallas_call(
        _c2_body,
        out_shape=jax.ShapeDtypeStruct((k, 1), jnp.float32),
    )(centroids)
    c2_row = c2_col.reshape(1, k)

    weights, assignments = pl.pallas_call(
        _router_body,
        grid=(n // TN,),
        in_specs=[
            pl.BlockSpec((TN, d), lambda i: (i, 0)),
            pl.BlockSpec((k, d), lambda i: (0, 0)),
            pl.BlockSpec((1, k), lambda i: (0, 0)),
        ],
        out_specs=[
            pl.BlockSpec((TN, k), lambda i: (i, 0)),
            pl.BlockSpec((TN, 1), lambda i: (i, 0)),
        ],
        out_shape=[
            jax.ShapeDtypeStruct((n, k), jnp.float32),
            jax.ShapeDtypeStruct((n, 1), jnp.int32),
        ],
    )(x_flat, c_bf16, c2_row)

    return weights.reshape(b, t, k), assignments.reshape(b, t)


# speedup vs baseline: 1.0423x; 1.0423x over previous
import jax
import jax.numpy as jnp
from jax.experimental import pallas as pl

TN = 1024
RG = 128


def _rowsum_sq(v):
    d = v.shape[1]
    vk = v[:, 0:128]
    q = vk * vk
    for k in range(1, d // 128):
        vk = v[:, 128 * k:128 * (k + 1)]
        q = q + vk * vk
    a = q[:, 0:8]
    for t in range(1, 16):
        a = a + q[:, 8 * t:8 * (t + 1)]
    b = a[:, 0:4] + a[:, 4:8]
    c = b[:, 0:2] + b[:, 2:4]
    return c[:, 0:1] + c[:, 1:2]


def _rowsum_sq_grouped(v):
    r = v.shape[0]
    if r <= RG:
        return _rowsum_sq(v)
    parts = [_rowsum_sq(v[i:i + RG, :]) for i in range(0, r, RG)]
    return jnp.concatenate(parts, axis=0)


def _c2_body(c_ref, o_ref):
    o_ref[...] = _rowsum_sq(c_ref[...])


def _router_body(x_ref, cb_ref, c2_ref, w_ref, a_ref):
    x = x_ref[...]
    cb = cb_ref[...]
    dot = jax.lax.dot_general(
        x, cb, (((1,), (1,)), ((), ())),
        preferred_element_type=jnp.float32,
    )
    x2 = _rowsum_sq(x)
    c2 = c2_ref[...]
    sq = jnp.maximum(x2 + c2 - 2.0 * dot, 0.0)
    dists = jnp.sqrt(sq)

    neg = -dists
    m = jnp.max(neg, axis=-1, keepdims=True)
    e = jnp.exp(neg - m)
    w_ref[...] = e / jnp.sum(e, axis=-1, keepdims=True)

    k = dists.shape[-1]
    idx = jax.lax.broadcasted_iota(jnp.int32, dists.shape, 1)
    minv = jnp.min(dists, axis=-1, keepdims=True)
    cand = jnp.where(dists == minv, idx, k)
    a_ref[...] = jnp.min(cand, axis=-1, keepdims=True)


def kernel(x, centroids):
    b, t, d = x.shape
    k = centroids.shape[0]
    n = b * t
    x_flat = x.reshape(n, d)
    c_bf16 = centroids.astype(jnp.bfloat16)

    c2_col = pl.pallas_call(
        _c2_body,
        out_shape=jax.ShapeDtypeStruct((k, 1), jnp.float32),
    )(centroids)
    c2_row = c2_col.reshape(1, k)

    weights, assignments = pl.pallas_call(
        _router_body,
        grid=(n // TN,),
        in_specs=[
            pl.BlockSpec((TN, d), lambda i: (i, 0)),
            pl.BlockSpec((k, d), lambda i: (0, 0)),
            pl.BlockSpec((1, k), lambda i: (0, 0)),
        ],
        out_specs=[
            pl.BlockSpec((TN, k), lambda i: (i, 0)),
            pl.BlockSpec((TN, 1), lambda i: (i, 0)),
        ],
        out_shape=[
            jax.ShapeDtypeStruct((n, k), jnp.float32),
            jax.ShapeDtypeStruct((n, 1), jnp.int32),
        ],
    )(x_flat, c_bf16, c2_row)

    return weights.reshape(b, t, k), assignments.reshape(b, t)


# speedup vs baseline: 1.0798x; 1.0360x over previous
import jax
import jax.numpy as jnp
from jax.experimental import pallas as pl
from jax.experimental.pallas import tpu as pltpu

TN = 1024
RG = 128


def _rowsum_sq(v):
    d = v.shape[1]
    vk = v[:, 0:128]
    q = vk * vk
    for k in range(1, d // 128):
        vk = v[:, 128 * k:128 * (k + 1)]
        q = q + vk * vk
    a = q[:, 0:8]
    for t in range(1, 16):
        a = a + q[:, 8 * t:8 * (t + 1)]
    b = a[:, 0:4] + a[:, 4:8]
    c = b[:, 0:2] + b[:, 2:4]
    return c[:, 0:1] + c[:, 1:2]


def _rowsum_sq_grouped(v):
    r = v.shape[0]
    if r <= RG:
        return _rowsum_sq(v)
    parts = [_rowsum_sq(v[i:i + RG, :]) for i in range(0, r, RG)]
    return jnp.concatenate(parts, axis=0)


def _c2_body(c_ref, o_ref):
    o_ref[...] = _rowsum_sq(c_ref[...])


def _router_body(x_ref, cb_ref, c2_ref, w_ref, a_ref):
    x = x_ref[...]
    cb = cb_ref[...]
    dot = jax.lax.dot_general(
        x, cb, (((1,), (1,)), ((), ())),
        preferred_element_type=jnp.float32,
    )
    x2 = _rowsum_sq(x)
    c2 = c2_ref[...]
    sq = jnp.maximum(x2 + c2 - 2.0 * dot, 0.0)
    dists = jnp.sqrt(sq)

    neg = -dists
    m = jnp.max(neg, axis=-1, keepdims=True)
    e = jnp.exp(neg - m)
    w_ref[...] = e / jnp.sum(e, axis=-1, keepdims=True)

    k = dists.shape[-1]
    idx = jax.lax.broadcasted_iota(jnp.int32, dists.shape, 1)
    minv = jnp.min(dists, axis=-1, keepdims=True)
    cand = jnp.where(dists == minv, idx, k)
    a_ref[...] = jnp.min(cand, axis=-1, keepdims=True)


def kernel(x, centroids):
    b, t, d = x.shape
    k = centroids.shape[0]
    n = b * t
    x_flat = x.reshape(n, d)

    c2_col = pl.pallas_call(
        _c2_body,
        out_shape=jax.ShapeDtypeStruct((k, 1), jnp.float32),
    )(centroids)
    c2_row = c2_col.reshape(1, k)

    weights, assignments = pl.pallas_call(
        _router_body,
        grid=(n // TN,),
        in_specs=[
            pl.BlockSpec((TN, d), lambda i: (i, 0)),
            pl.BlockSpec((k, d), lambda i: (0, 0)),
            pl.BlockSpec((1, k), lambda i: (0, 0)),
        ],
        out_specs=[
            pl.BlockSpec((TN, k), lambda i: (i, 0)),
            pl.BlockSpec((TN, 1), lambda i: (i, 0)),
        ],
        out_shape=[
            jax.ShapeDtypeStruct((n, k), jnp.float32),
            jax.ShapeDtypeStruct((n, 1), jnp.int32),
        ],
        compiler_params=pltpu.CompilerParams(
            dimension_semantics=("parallel",),
        ),
    )(x_flat, centroids, c2_row)

    return weights.reshape(b, t, k), assignments.reshape(b, t)
